# Initial kernel scaffold; baseline (speedup 1.0000x reference)
import jax
import jax.numpy as jnp
from jax.experimental import pallas as pl


def kernel(x, edge_index, batch, W1, a_src1, a_dst1, b1, W2, a_src2, a_dst2, b2):
    raise NotImplementedError("write your pallas kernel here")


# speedup vs baseline: 17.1567x; 17.1567x over previous
import functools

import jax
import jax.numpy as jnp
from jax import lax
from jax.experimental import pallas as pl
from jax.experimental.pallas import tpu as pltpu
from jax.experimental.pallas import tpu_sc as plsc

F32 = jnp.float32
I32 = jnp.int32

NTILE = 16
NCORE = 2
K = 128


def _ceil_to(a, m):
    return (a + m - 1) // m * m


def _proj1_body(x_ref, w_ref, af_ref, df_ref, hp_ref, asad_ref):
    h = jnp.dot(x_ref[...], w_ref[...], preferred_element_type=F32)
    r = h.shape[0]
    hp_ref[0] = h[:, :128]
    hp_ref[1] = h[:, 128:]
    hs = (h * af_ref[...]).reshape(r, 4, 64).sum(-1)
    hd = (h * df_ref[...]).reshape(r, 4, 64).sum(-1)
    asad_ref[...] = jnp.concatenate([hs, hd], axis=1)


def _proj1(x, W1, af, df, n, r):
    grid = n // r
    return pl.pallas_call(
        _proj1_body,
        grid=(grid,),
        in_specs=[
            pl.BlockSpec((r, x.shape[1]), lambda i: (i, 0)),
            pl.BlockSpec(W1.shape, lambda i: (0, 0)),
            pl.BlockSpec((1, 256), lambda i: (0, 0)),
            pl.BlockSpec((1, 256), lambda i: (0, 0)),
        ],
        out_specs=[
            pl.BlockSpec((2, r, 128), lambda i: (0, i, 0)),
            pl.BlockSpec((r, 8), lambda i: (i, 0)),
        ],
        out_shape=[
            jax.ShapeDtypeStruct((2, n, 128), F32),
            jax.ShapeDtypeStruct((n + 1, 8), F32),
        ],
    )(x, W1, af, df)


def _make_sc_ex(n, e2p, NH):
    T32 = e2p // (NCORE * NTILE)
    nchunk = T32 // K
    mesh = plsc.VectorSubcoreMesh(core_axis_name="c", subcore_axis_name="s")

    def body(asad_hbm, srcp, dstp, ex_hbm, asad_v, exbuf, src_v, dst_v):
        core = lax.axis_index("c")
        s = lax.axis_index("s")
        w = core * NTILE + s
        pltpu.sync_copy(asad_hbm, asad_v)
        iot = lax.iota(I32, 16)

        def chunk(c, _):
            e0 = w * T32 + c * K
            pltpu.sync_copy(srcp.at[pl.ds(e0, K)], src_v)
            pltpu.sync_copy(dstp.at[pl.ds(e0, K)], dst_v)
            for v in range(K // 16):
                srcv = src_v[pl.ds(v * 16, 16)]
                dstv = dst_v[pl.ds(v * 16, 16)]
                rows = iot + v * 16
                for hg in range(NH):
                    sa = plsc.load_gather(asad_v, [srcv, jnp.full((16,), hg, I32)])
                    da = plsc.load_gather(asad_v, [dstv, jnp.full((16,), NH + hg, I32)])
                    t = sa + da
                    ex = jnp.exp(jnp.maximum(t, 0.2 * t))
                    plsc.store_scatter(exbuf, [jnp.full((16,), hg, I32), rows], ex)
            for hg in range(NH):
                pltpu.sync_copy(exbuf.at[hg, pl.ds(0, K)],
                                ex_hbm.at[hg, pl.ds(e0, K)])
            return 0

        lax.fori_loop(0, nchunk, chunk, 0)

    return pl.kernel(
        body,
        out_type=jax.ShapeDtypeStruct((NH, e2p), F32),
        mesh=mesh,
        scratch_types=[
            pltpu.VMEM((n + 1, 2 * NH), F32),
            pltpu.VMEM((NH, K), F32),
            pltpu.VMEM((K,), I32),
            pltpu.VMEM((K,), I32),
        ],
        compiler_params=pltpu.CompilerParams(
            needs_layout_passes=False, use_tc_tiling_on_sc=False),
    )


def _make_sc_msg(n, e2p, F, HPC, NH, COLS):
    n_acc = _ceil_to(n + 1, NTILE * 8)
    zrows = n_acc // NTILE
    T = e2p // NTILE
    nchunk = T // K
    NQ = F // 16
    mesh = plsc.VectorSubcoreMesh(core_axis_name="c", subcore_axis_name="s")

    def body(hflat, ex_hbm, srcp, dstp, msg_hbm,
             msg_acc, grows, comb, exh, src_v, dst_v, sadj_v, sem):
        core = lax.axis_index("c")
        s = lax.axis_index("s")

        zv = jnp.zeros((16,), F32)

        def zloop(i, _):
            for q in range(COLS // 16):
                comb[i, pl.ds(q * 16, 16)] = zv
            return 0

        lax.fori_loop(0, K, zloop, 0)
        z0 = s * zrows
        full, rem = zrows // K, zrows % K
        for kk in range(full):
            pltpu.sync_copy(comb, msg_acc.at[pl.ds(z0 + kk * K, K)])
        if rem:
            pltpu.sync_copy(comb.at[pl.ds(0, rem)],
                            msg_acc.at[pl.ds(z0 + full * K, rem)])
        plsc.subcore_barrier()

        iot = lax.iota(I32, 16)

        def chunk(c, _):
            e0 = s * T + c * K
            pltpu.sync_copy(srcp.at[pl.ds(e0, K)], src_v)
            pltpu.sync_copy(dstp.at[pl.ds(e0, K)], dst_v)
            for v in range(K // 16):
                sadj_v[pl.ds(v * 16, 16)] = src_v[pl.ds(v * 16, 16)] + core * n
            gcp = pltpu.async_copy(hflat.at[sadj_v], grows, sem)
            for h in range(HPC):
                hg = core * HPC + h if NH == NCORE * HPC else h
                pltpu.sync_copy(ex_hbm.at[hg, pl.ds(e0, K)],
                                exh.at[h, pl.ds(0, K)])
            for v in range(K // 16):
                rows = iot + v * 16
                for h in range(HPC):
                    plsc.store_scatter(comb, [rows, jnp.full((16,), F + h, I32)],
                                       exh[h, pl.ds(v * 16, 16)])
            gcp.wait()

            def scale(e, _):
                exs = [exh[h, pl.ds(e, 16)][0] for h in range(HPC)]
                for q in range(NQ):
                    ex = exs[q // (NQ // HPC)]
                    comb[e, pl.ds(q * 16, 16)] = grows[e, pl.ds(q * 16, 16)] * ex
                return 0

            lax.fori_loop(0, K, scale, 0, unroll=4)
            pltpu.sync_copy(comb, msg_acc.at[dst_v], add=True)
            return 0

        lax.fori_loop(0, nchunk, chunk, 0)
        plsc.subcore_barrier()
        pltpu.sync_copy(msg_acc.at[pl.ds(z0, zrows)],
                        msg_hbm.at[core, pl.ds(z0, zrows)])

    return pl.kernel(
        body,
        out_type=jax.ShapeDtypeStruct((NCORE, n_acc, COLS), F32),
        mesh=mesh,
        scratch_types=[
            pltpu.VMEM_SHARED((n_acc, COLS), F32),
            pltpu.VMEM((K, F), F32),
            pltpu.VMEM((K, COLS), F32),
            pltpu.VMEM((HPC, K + 16), F32),
            pltpu.VMEM((K,), I32),
            pltpu.VMEM((K,), I32),
            pltpu.VMEM((K,), I32),
            pltpu.SemaphoreType.DMA,
        ],
        compiler_params=pltpu.CompilerParams(
            needs_layout_passes=False, use_tc_tiling_on_sc=False),
    )


def _mid_body(msg_ref, w2_ref, b1_ref, a2_ref, hp_ref, asad_ref):
    m0 = msg_ref[0]
    m1 = msg_ref[1]
    eps = jnp.float32(1e-16)
    h1 = jnp.concatenate([
        m0[:, 0:64] / (m0[:, 128:129] + eps),
        m0[:, 64:128] / (m0[:, 129:130] + eps),
        m1[:, 0:64] / (m1[:, 128:129] + eps),
        m1[:, 64:128] / (m1[:, 129:130] + eps),
    ], axis=1) + b1_ref[...]
    h1 = jnp.where(h1 > 0, h1, jnp.exp(h1) - 1.0)
    hc = jnp.dot(h1, w2_ref[...], preferred_element_type=F32)
    hp_ref[0] = hc[:, :32]
    hp_ref[1] = hc[:, 32:]
    asad_ref[...] = jnp.dot(hc, a2_ref[...], preferred_element_type=F32)


def _mid(msg1, W2, b1, a2, n, r):
    grid = n // r
    return pl.pallas_call(
        _mid_body,
        grid=(grid,),
        in_specs=[
            pl.BlockSpec((2, r, 144), lambda i: (0, i, 0)),
            pl.BlockSpec(W2.shape, lambda i: (0, 0)),
            pl.BlockSpec((1, 256), lambda i: (0, 0)),
            pl.BlockSpec((64, 2), lambda i: (0, 0)),
        ],
        out_specs=[
            pl.BlockSpec((2, r, 32), lambda i: (0, i, 0)),
            pl.BlockSpec((r, 2), lambda i: (i, 0)),
        ],
        out_shape=[
            jax.ShapeDtypeStruct((2, n, 32), F32),
            jax.ShapeDtypeStruct((n + 1, 2), F32),
        ],
    )(msg1, W2, b1, a2)


def _pool_body(msg_ref, b2_ref, batch_ref, out_ref, acc_ref):
    i = pl.program_id(0)
    m0 = msg_ref[0]
    m1 = msg_ref[1]
    eps = jnp.float32(1e-16)
    den = m0[:, 32:33] + eps
    h2 = jnp.concatenate([m0[:, :32] / den, m1[:, :32] / den], axis=1) \
        + b2_ref[...]
    h2 = jnp.where(h2 > 0, h2, jnp.exp(h2) - 1.0)
    r = h2.shape[0]
    hcat = jnp.concatenate([h2, jnp.ones((r, 1), F32)], axis=1)
    b = batch_ref[0, 0, :]
    onehot = (b[:, None] == lax.broadcasted_iota(I32, (1, 16), 1)).astype(F32)
    contrib = jnp.dot(onehot.T, hcat, preferred_element_type=F32)

    @pl.when(i == 0)
    def _():
        acc_ref[...] = contrib

    @pl.when(i > 0)
    def _():
        acc_ref[...] += contrib

    @pl.when(i == pl.num_programs(0) - 1)
    def _():
        a = acc_ref[...]
        out_ref[...] = a[:, :64] / jnp.maximum(a[:, 64:65], 1.0)


def _pool(msg2, b2, batch3, n, r):
    grid = n // r
    return pl.pallas_call(
        _pool_body,
        grid=(grid,),
        in_specs=[
            pl.BlockSpec((2, r, 48), lambda i: (0, i, 0)),
            pl.BlockSpec((1, 64), lambda i: (0, 0)),
            pl.BlockSpec((1, 1, r), lambda i: (i, 0, 0)),
        ],
        out_specs=pl.BlockSpec((16, 64), lambda i: (0, 0)),
        out_shape=jax.ShapeDtypeStruct((16, 64), F32),
        scratch_shapes=[pltpu.VMEM((16, 65), F32)],
    )(msg2, b2, batch3)


def kernel(x, edge_index, batch, W1, a_src1, a_dst1, b1, W2, a_src2, a_dst2, b2):
    n, d = x.shape
    e = edge_index.shape[1]
    r = 1000

    loop = jnp.arange(n, dtype=edge_index.dtype)
    e2 = e + n
    e2p = _ceil_to(e2, NTILE * K)
    pad = e2p - e2
    srcp = jnp.concatenate([edge_index[0], loop, jnp.zeros((pad,), I32)])
    dstp = jnp.concatenate([edge_index[1], loop, jnp.full((pad,), n, I32)])
    af = a_src1.reshape(1, 256)
    df = a_dst1.reshape(1, 256)
    a2 = jnp.stack([a_src2[0], a_dst2[0]], axis=1)
    batch3 = batch.astype(I32).reshape(n // r, 1, r)

    hpair, asad1 = _proj1(x, W1, af, df, n, r)
    ex1 = _make_sc_ex(n, e2p, 4)(asad1, srcp, dstp)
    msg1 = _make_sc_msg(n, e2p, 128, 2, 4, 144)(
        hpair.reshape(2 * n, 128), ex1, srcp, dstp)

    h2pair, asad2 = _mid(msg1, W2, b1.reshape(1, 256), a2, n, r)
    ex2 = _make_sc_ex(n, e2p, 1)(asad2, srcp, dstp)
    msg2 = _make_sc_msg(n, e2p, 32, 1, 1, 48)(
        h2pair.reshape(2 * n, 32), ex2, srcp, dstp)

    return _pool(msg2, b2.reshape(1, 64), batch3, n, r)


# speedup vs baseline: 17.9185x; 1.0444x over previous
import functools

import jax
import jax.numpy as jnp
from jax import lax
from jax.experimental import pallas as pl
from jax.experimental.pallas import tpu as pltpu
from jax.experimental.pallas import tpu_sc as plsc

F32 = jnp.float32
I32 = jnp.int32

NTILE = 16
NCORE = 2
K = 128


def _ceil_to(a, m):
    return (a + m - 1) // m * m


def _proj1_body(x_ref, w_ref, af_ref, df_ref, hp_ref, asad_ref):
    h = jnp.dot(x_ref[...], w_ref[...], preferred_element_type=F32)
    r = h.shape[0]
    hp_ref[0] = h[:, :128]
    hp_ref[1] = h[:, 128:]
    hs = (h * af_ref[...]).reshape(r, 4, 64).sum(-1)
    hd = (h * df_ref[...]).reshape(r, 4, 64).sum(-1)
    asad_ref[...] = jnp.concatenate([hs, hd], axis=1)


def _proj1(x, W1, af, df, n, r):
    grid = n // r
    return pl.pallas_call(
        _proj1_body,
        grid=(grid,),
        in_specs=[
            pl.BlockSpec((r, x.shape[1]), lambda i: (i, 0)),
            pl.BlockSpec(W1.shape, lambda i: (0, 0)),
            pl.BlockSpec((1, 256), lambda i: (0, 0)),
            pl.BlockSpec((1, 256), lambda i: (0, 0)),
        ],
        out_specs=[
            pl.BlockSpec((2, r, 128), lambda i: (0, i, 0)),
            pl.BlockSpec((r, 8), lambda i: (i, 0)),
        ],
        out_shape=[
            jax.ShapeDtypeStruct((2, n, 128), F32),
            jax.ShapeDtypeStruct((n + 1, 8), F32),
        ],
    )(x, W1, af, df)


def _make_sc_ex(n, e2p, NH):
    T32 = e2p // (NCORE * NTILE)
    nchunk = T32 // K
    mesh = plsc.VectorSubcoreMesh(core_axis_name="c", subcore_axis_name="s")

    def body(asad_hbm, srcp, dstp, ex_hbm, asad_v, exbuf, src_v, dst_v):
        core = lax.axis_index("c")
        s = lax.axis_index("s")
        w = core * NTILE + s
        pltpu.sync_copy(asad_hbm, asad_v)
        iot = lax.iota(I32, 16)

        def chunk(c, _):
            e0 = w * T32 + c * K
            pltpu.sync_copy(srcp.at[pl.ds(e0, K)], src_v)
            pltpu.sync_copy(dstp.at[pl.ds(e0, K)], dst_v)
            for v in range(K // 16):
                srcv = src_v[pl.ds(v * 16, 16)]
                dstv = dst_v[pl.ds(v * 16, 16)]
                rows = iot + v * 16
                for hg in range(NH):
                    sa = plsc.load_gather(asad_v, [srcv, jnp.full((16,), hg, I32)])
                    da = plsc.load_gather(asad_v, [dstv, jnp.full((16,), NH + hg, I32)])
                    t = sa + da
                    ex = jnp.exp(jnp.maximum(t, 0.2 * t))
                    plsc.store_scatter(exbuf, [jnp.full((16,), hg, I32), rows], ex)
            for hg in range(NH):
                pltpu.sync_copy(exbuf.at[hg, pl.ds(0, K)],
                                ex_hbm.at[hg, pl.ds(e0, K)])
            return 0

        lax.fori_loop(0, nchunk, chunk, 0)

    return pl.kernel(
        body,
        out_type=jax.ShapeDtypeStruct((NH, e2p), F32),
        mesh=mesh,
        scratch_types=[
            pltpu.VMEM((n + 1, 2 * NH), F32),
            pltpu.VMEM((NH, K), F32),
            pltpu.VMEM((K,), I32),
            pltpu.VMEM((K,), I32),
        ],
        compiler_params=pltpu.CompilerParams(
            needs_layout_passes=False, use_tc_tiling_on_sc=False),
    )


def _make_sc_msg(n, e2p, F, HPC, NH, COLS):
    n_acc = _ceil_to(n + 1, NTILE * 8)
    zrows = n_acc // NTILE
    T = e2p // NTILE
    nchunk = T // K
    NQ = F // 16
    mesh = plsc.VectorSubcoreMesh(core_axis_name="c", subcore_axis_name="s")

    def body(hflat, ex_hbm, srcp, dstp, msg_hbm,
             msg_acc, grows, comb, exh, src_v, dst_v, sadj_v, sem):
        core = lax.axis_index("c")
        s = lax.axis_index("s")

        zv = jnp.zeros((16,), F32)

        def zloop(i, _):
            for q in range(COLS // 16):
                comb[i, pl.ds(q * 16, 16)] = zv
            return 0

        lax.fori_loop(0, K, zloop, 0)
        z0 = s * zrows
        full, rem = zrows // K, zrows % K
        for kk in range(full):
            pltpu.sync_copy(comb, msg_acc.at[pl.ds(z0 + kk * K, K)])
        if rem:
            pltpu.sync_copy(comb.at[pl.ds(0, rem)],
                            msg_acc.at[pl.ds(z0 + full * K, rem)])
        plsc.subcore_barrier()

        iot = lax.iota(I32, 16)

        def _prep_gather(e0):
            pltpu.sync_copy(srcp.at[pl.ds(e0, K)], src_v)
            for v in range(K // 16):
                sadj_v[pl.ds(v * 16, 16)] = src_v[pl.ds(v * 16, 16)] + core * n
            pltpu.async_copy(hflat.at[sadj_v], grows, sem)

        _prep_gather(s * T)
        pltpu.sync_copy(dstp.at[pl.ds(s * T, K)], dst_v)

        def chunk(c, _):
            e0 = s * T + c * K
            for h in range(HPC):
                hg = core * HPC + h if NH == NCORE * HPC else h
                pltpu.sync_copy(ex_hbm.at[hg, pl.ds(e0, K)],
                                exh.at[h, pl.ds(0, K)])
            for v in range(K // 16):
                rows = iot + v * 16
                for h in range(HPC):
                    plsc.store_scatter(comb, [rows, jnp.full((16,), F + h, I32)],
                                       exh[h, pl.ds(v * 16, 16)])
            pltpu.make_async_copy(hflat.at[sadj_v], grows, sem).wait()

            def scale(e, _):
                exs = [exh[h, pl.ds(e, 16)][0] for h in range(HPC)]
                for q in range(NQ):
                    ex = exs[q // (NQ // HPC)]
                    comb[e, pl.ds(q * 16, 16)] = grows[e, pl.ds(q * 16, 16)] * ex
                return 0

            lax.fori_loop(0, K, scale, 0, unroll=4)

            @pl.when(c < nchunk - 1)
            def _():
                _prep_gather(e0 + K)

            pltpu.sync_copy(comb, msg_acc.at[dst_v], add=True)

            @pl.when(c < nchunk - 1)
            def _():
                pltpu.sync_copy(dstp.at[pl.ds(e0 + K, K)], dst_v)
            return 0

        lax.fori_loop(0, nchunk, chunk, 0)
        plsc.subcore_barrier()
        pltpu.sync_copy(msg_acc.at[pl.ds(z0, zrows)],
                        msg_hbm.at[core, pl.ds(z0, zrows)])

    return pl.kernel(
        body,
        out_type=jax.ShapeDtypeStruct((NCORE, n_acc, COLS), F32),
        mesh=mesh,
        scratch_types=[
            pltpu.VMEM_SHARED((n_acc, COLS), F32),
            pltpu.VMEM((K, F), F32),
            pltpu.VMEM((K, COLS), F32),
            pltpu.VMEM((HPC, K + 16), F32),
            pltpu.VMEM((K,), I32),
            pltpu.VMEM((K,), I32),
            pltpu.VMEM((K,), I32),
            pltpu.SemaphoreType.DMA,
        ],
        compiler_params=pltpu.CompilerParams(
            needs_layout_passes=False, use_tc_tiling_on_sc=False),
    )


def _mid_body(msg_ref, w2_ref, b1_ref, a2_ref, hp_ref, asad_ref):
    m0 = msg_ref[0]
    m1 = msg_ref[1]
    eps = jnp.float32(1e-16)
    h1 = jnp.concatenate([
        m0[:, 0:64] / (m0[:, 128:129] + eps),
        m0[:, 64:128] / (m0[:, 129:130] + eps),
        m1[:, 0:64] / (m1[:, 128:129] + eps),
        m1[:, 64:128] / (m1[:, 129:130] + eps),
    ], axis=1) + b1_ref[...]
    h1 = jnp.where(h1 > 0, h1, jnp.exp(h1) - 1.0)
    hc = jnp.dot(h1, w2_ref[...], preferred_element_type=F32)
    hp_ref[0] = hc[:, :32]
    hp_ref[1] = hc[:, 32:]
    asad_ref[...] = jnp.dot(hc, a2_ref[...], preferred_element_type=F32)


def _mid(msg1, W2, b1, a2, n, r):
    grid = n // r
    return pl.pallas_call(
        _mid_body,
        grid=(grid,),
        in_specs=[
            pl.BlockSpec((2, r, 144), lambda i: (0, i, 0)),
            pl.BlockSpec(W2.shape, lambda i: (0, 0)),
            pl.BlockSpec((1, 256), lambda i: (0, 0)),
            pl.BlockSpec((64, 2), lambda i: (0, 0)),
        ],
        out_specs=[
            pl.BlockSpec((2, r, 32), lambda i: (0, i, 0)),
            pl.BlockSpec((r, 2), lambda i: (i, 0)),
        ],
        out_shape=[
            jax.ShapeDtypeStruct((2, n, 32), F32),
            jax.ShapeDtypeStruct((n + 1, 2), F32),
        ],
    )(msg1, W2, b1, a2)


def _pool_body(msg_ref, b2_ref, batch_ref, out_ref, acc_ref):
    i = pl.program_id(0)
    m0 = msg_ref[0]
    m1 = msg_ref[1]
    eps = jnp.float32(1e-16)
    den = m0[:, 32:33] + eps
    h2 = jnp.concatenate([m0[:, :32] / den, m1[:, :32] / den], axis=1) \
        + b2_ref[...]
    h2 = jnp.where(h2 > 0, h2, jnp.exp(h2) - 1.0)
    r = h2.shape[0]
    hcat = jnp.concatenate([h2, jnp.ones((r, 1), F32)], axis=1)
    b = batch_ref[0, 0, :]
    onehot = (b[:, None] == lax.broadcasted_iota(I32, (1, 16), 1)).astype(F32)
    contrib = jnp.dot(onehot.T, hcat, preferred_element_type=F32)

    @pl.when(i == 0)
    def _():
        acc_ref[...] = contrib

    @pl.when(i > 0)
    def _():
        acc_ref[...] += contrib

    @pl.when(i == pl.num_programs(0) - 1)
    def _():
        a = acc_ref[...]
        out_ref[...] = a[:, :64] / jnp.maximum(a[:, 64:65], 1.0)


def _pool(msg2, b2, batch3, n, r):
    grid = n // r
    return pl.pallas_call(
        _pool_body,
        grid=(grid,),
        in_specs=[
            pl.BlockSpec((2, r, 48), lambda i: (0, i, 0)),
            pl.BlockSpec((1, 64), lambda i: (0, 0)),
            pl.BlockSpec((1, 1, r), lambda i: (i, 0, 0)),
        ],
        out_specs=pl.BlockSpec((16, 64), lambda i: (0, 0)),
        out_shape=jax.ShapeDtypeStruct((16, 64), F32),
        scratch_shapes=[pltpu.VMEM((16, 65), F32)],
    )(msg2, b2, batch3)


def kernel(x, edge_index, batch, W1, a_src1, a_dst1, b1, W2, a_src2, a_dst2, b2):
    n, d = x.shape
    e = edge_index.shape[1]
    r = 1000

    loop = jnp.arange(n, dtype=edge_index.dtype)
    e2 = e + n
    e2p = _ceil_to(e2, NTILE * K)
    pad = e2p - e2
    srcp = jnp.concatenate([edge_index[0], loop, jnp.zeros((pad,), I32)])
    dstp = jnp.concatenate([edge_index[1], loop, jnp.full((pad,), n, I32)])
    af = a_src1.reshape(1, 256)
    df = a_dst1.reshape(1, 256)
    a2 = jnp.stack([a_src2[0], a_dst2[0]], axis=1)
    batch3 = batch.astype(I32).reshape(n // r, 1, r)

    hpair, asad1 = _proj1(x, W1, af, df, n, r)
    ex1 = _make_sc_ex(n, e2p, 4)(asad1, srcp, dstp)
    msg1 = _make_sc_msg(n, e2p, 128, 2, 4, 144)(
        hpair.reshape(2 * n, 128), ex1, srcp, dstp)

    h2pair, asad2 = _mid(msg1, W2, b1.reshape(1, 256), a2, n, r)
    ex2 = _make_sc_ex(n, e2p, 1)(asad2, srcp, dstp)
    msg2 = _make_sc_msg(n, e2p, 32, 1, 1, 48)(
        h2pair.reshape(2 * n, 32), ex2, srcp, dstp)

    return _pool(msg2, b2.reshape(1, 64), batch3, n, r)
